# Initial kernel scaffold; baseline (speedup 1.0000x reference)
import jax
import jax.numpy as jnp
from jax.experimental import pallas as pl


def kernel(nodes, neigh_idx, feat_table, W, gamma, beta):
    raise NotImplementedError("write your pallas kernel here")


# speedup vs baseline: 1.4891x; 1.4891x over previous
import functools

import jax
import jax.numpy as jnp
from jax import lax
from jax.experimental import pallas as pl
from jax.experimental.pallas import tpu as pltpu
from jax.experimental.pallas import tpu_sc as plsc

D = 128
S = 10
NC = 2
NS = 16
NW = NC * NS
B_PAD = 51200
R = B_PAD // NW
C = 64
NCHUNK = R // C


def _sc_gather_pool(nodes_hbm, neigh_hbm, feat_hbm, self_out, neigh_out,
                    sidx, nidx, srows, nbuf, nmean, sem):
    wid = lax.axis_index("s") * NC + lax.axis_index("c")
    base = wid * R

    def chunk_body(j, carry):
        cb = pl.multiple_of(base + j * C, C)
        pltpu.sync_copy(nodes_hbm.at[pl.ds(cb, C)], sidx)
        pltpu.sync_copy(neigh_hbm.at[pl.ds(cb * S, C * S)], nidx)
        cps = [pltpu.async_copy(feat_hbm.at[sidx], srows, sem)]
        for k in range(C * S // 128):
            cps.append(pltpu.async_copy(
                feat_hbm.at[nidx.at[pl.ds(k * 128, 128)]],
                nbuf.at[pl.ds(k * 128, 128)], sem))
        for cp in cps:
            cp.wait()

        def row_body(c, rc):
            r0 = c * S
            for v in range(D // 16):
                col = pl.ds(v * 16, 16)
                acc = nbuf[r0, col]
                for s in range(1, S):
                    acc = acc + nbuf[r0 + s, col]
                nmean[c, col] = acc * (1.0 / S)
            return rc

        lax.fori_loop(0, C, row_body, 0)
        pltpu.sync_copy(srows, self_out.at[pl.ds(cb, C)])
        pltpu.sync_copy(nmean, neigh_out.at[pl.ds(cb, C)])
        return carry

    lax.fori_loop(0, NCHUNK, chunk_body, 0)


def _sc_gather(nodes_p, neigh_p, feat_table):
    mesh = plsc.VectorSubcoreMesh(core_axis_name="c", subcore_axis_name="s")
    fn = functools.partial(
        pl.kernel,
        mesh=mesh,
        out_type=[
            jax.ShapeDtypeStruct((B_PAD, D), jnp.float32),
            jax.ShapeDtypeStruct((B_PAD, D), jnp.float32),
        ],
        scratch_types=[
            pltpu.VMEM((C,), jnp.int32),
            pltpu.VMEM((C * S,), jnp.int32),
            pltpu.VMEM((C, D), jnp.float32),
            pltpu.VMEM((C * S, D), jnp.float32),
            pltpu.VMEM((C, D), jnp.float32),
            pltpu.SemaphoreType.DMA,
        ],
    )(_sc_gather_pool)
    return fn(nodes_p, neigh_p, feat_table)


def _stats_body(x_ref, o_ref):
    @pl.when(pl.program_id(0) == 0)
    def _():
        o_ref[...] = jnp.zeros_like(o_ref)

    x = x_ref[...]
    o_ref[...] += jnp.concatenate(
        [jnp.sum(x, 0)[None, :], jnp.sum(x * x, 0)[None, :]], axis=0)


def _mm_body(nbatch, self_ref, neigh_ref, w_ref, p_ref, o_ref):
    p = p_ref[...]
    mu = p[0] / nbatch
    var = p[1] / nbatch - mu * mu
    scale = p[2] * lax.rsqrt(var + 1e-5)
    bias = p[3] - mu * scale
    s_norm = self_ref[...] * scale[None, :] + bias[None, :]
    w = w_ref[...]
    o = lax.dot_general(w[:, :D], s_norm, (((1,), (1,)), ((), ())),
                        precision=lax.Precision.HIGHEST)
    o = o + lax.dot_general(w[:, D:], neigh_ref[...], (((1,), (1,)), ((), ())),
                            precision=lax.Precision.HIGHEST)
    o_ref[...] = jnp.maximum(o, 0.0)


def kernel(nodes, neigh_idx, feat_table, W, gamma, beta):
    B = nodes.shape[0]
    E = W.shape[0]
    nodes_p = jnp.pad(nodes.astype(jnp.int32), (0, B_PAD - B))
    neigh_p = jnp.pad(neigh_idx.astype(jnp.int32).reshape(-1),
                      (0, (B_PAD - B) * S))
    feat_table = feat_table.astype(jnp.float32)

    self_raw, neigh_mean = _sc_gather(nodes_p, neigh_p, feat_table)

    rows_blk = 1000
    assert B % rows_blk == 0
    stats = pl.pallas_call(
        _stats_body,
        grid=(B // rows_blk,),
        in_specs=[pl.BlockSpec((rows_blk, D), lambda i: (i, 0))],
        out_specs=pl.BlockSpec((2, D), lambda i: (0, 0)),
        out_shape=jax.ShapeDtypeStruct((2, D), jnp.float32),
    )(self_raw)

    params = jnp.concatenate(
        [stats, gamma[None, :].astype(jnp.float32),
         beta[None, :].astype(jnp.float32)], axis=0)

    bn = 512
    out = pl.pallas_call(
        functools.partial(_mm_body, float(B)),
        grid=(B_PAD // bn,),
        in_specs=[
            pl.BlockSpec((bn, D), lambda i: (i, 0)),
            pl.BlockSpec((bn, D), lambda i: (i, 0)),
            pl.BlockSpec((E, 2 * D), lambda i: (0, 0)),
            pl.BlockSpec((4, D), lambda i: (0, 0)),
        ],
        out_specs=pl.BlockSpec((E, bn), lambda i: (0, i)),
        out_shape=jax.ShapeDtypeStruct((E, B_PAD), jnp.float32),
    )(self_raw, neigh_mean, W.astype(jnp.float32), params)

    return out[:, :B]
